# baseline (speedup 1.0000x reference)
import functools

import jax
import jax.numpy as jnp
from jax.experimental import pallas as pl
from jax.experimental.pallas import tpu as pltpu

_K = 32
_T = 128
_C = 512


def _fused_kernel(x_ref, wenc_ref, benc_ref, wdec_ref, bdec_ref,
                  y_ref, psum_ref, z_ref, *, n_chunks, k):
    j = pl.program_id(1)
    x = x_ref[...]
    zc = jnp.dot(x, wenc_ref[...], preferred_element_type=jnp.float32)
    zc = jnp.maximum(zc + benc_ref[...], 0.0)
    z_ref[:, pl.ds(j * _C, _C)] = zc

    @pl.when(j == n_chunks - 1)
    def _finish():
        z = z_ref[...]
        t_rows = z.shape[0]
        kf = jnp.float32(k)

        def count_ge(thr):
            return jnp.sum((z >= thr).astype(jnp.float32), axis=1,
                           keepdims=True)

        def p1_body(i, bits):
            sh = 29 - 2 * i
            f = lambda c: jax.lax.bitcast_convert_type(c, jnp.float32)
            c1 = bits | (jnp.int32(1) << sh)
            c2 = bits | (jnp.int32(2) << sh)
            c3 = bits | (jnp.int32(3) << sh)
            j = ((count_ge(f(c1)) >= kf).astype(jnp.int32)
                 + (count_ge(f(c2)) >= kf).astype(jnp.int32)
                 + (count_ge(f(c3)) >= kf).astype(jnp.int32))
            return bits | (j << sh)

        bits0 = jnp.zeros((t_rows, 1), jnp.int32)
        bits = jax.lax.fori_loop(0, 8, p1_body, bits0)
        t0 = jax.lax.bitcast_convert_type(bits, jnp.float32)
        c0 = count_ge(t0)

        def p2_cond(state):
            _, c, done = state
            live = jnp.where((done < 0.5) & (c > kf), 1.0, 0.0)
            return jnp.max(live) > 0.5

        def p2_body(state):
            t, c, done = state
            mn = jnp.min(jnp.where(z >= t, z, jnp.float32(jnp.inf)),
                         axis=1, keepdims=True)
            nxt = jax.lax.bitcast_convert_type(
                jax.lax.bitcast_convert_type(mn, jnp.int32) + 1,
                jnp.float32)
            c2 = count_ge(nxt)
            grp_done = c2 < kf
            active = (done < 0.5) & (c > kf)
            t_new = jnp.where(active, jnp.where(grp_done, mn, nxt), t)
            c_new = jnp.where(active & jnp.logical_not(grp_done), c2, c)
            done_new = jnp.where(active & grp_done, 1.0, done)
            return (t_new, c_new, done_new)

        thr, _, _ = jax.lax.while_loop(
            p2_cond, p2_body,
            (t0, c0, jnp.zeros((t_rows, 1), jnp.float32)))

        z_sparse = jnp.where(z >= thr, z, 0.0)
        y = jnp.dot(z_sparse, wdec_ref[...],
                    preferred_element_type=jnp.float32) + bdec_ref[...]
        y_ref[...] = y
        d = y - x
        psum_ref[...] = jnp.broadcast_to(
            jnp.sum(d * d, keepdims=True).reshape(1, 1, 1), (1, 1, 128))


def kernel(inputs_embeds, W_enc, b_enc, W_dec, b_dec):
    bsz, seq, dim = inputs_embeds.shape
    code = W_enc.shape[1]
    tokens = bsz * seq
    n_blocks = tokens // _T
    n_chunks = code // _C

    x2 = inputs_embeds.reshape(tokens, dim)
    benc2 = b_enc.reshape(1, code)
    bdec2 = b_dec.reshape(1, dim)

    grid = (n_blocks, n_chunks)
    y, psums = pl.pallas_call(
        functools.partial(_fused_kernel, n_chunks=n_chunks, k=_K),
        grid=grid,
        in_specs=[
            pl.BlockSpec((_T, dim), lambda i, j: (i, 0)),
            pl.BlockSpec((dim, _C), lambda i, j: (0, j)),
            pl.BlockSpec((1, _C), lambda i, j: (0, j)),
            pl.BlockSpec((code, dim), lambda i, j: (0, 0)),
            pl.BlockSpec((1, dim), lambda i, j: (0, 0)),
        ],
        out_specs=[
            pl.BlockSpec((_T, dim), lambda i, j: (i, 0)),
            pl.BlockSpec((1, 1, 128), lambda i, j: (i, 0, 0)),
        ],
        out_shape=[
            jax.ShapeDtypeStruct((tokens, dim), jnp.float32),
            jax.ShapeDtypeStruct((n_blocks, 1, 128), jnp.float32),
        ],
        scratch_shapes=[pltpu.VMEM((_T, code), jnp.float32)],
        compiler_params=pltpu.CompilerParams(
            dimension_semantics=("parallel", "arbitrary"),
        ),
    )(x2, W_enc, benc2, W_dec, bdec2)

    output_embeds = y.reshape(bsz, seq, dim)
    vq_loss = 0.25 * jnp.sum(psums[:, 0, 0]) / (tokens * dim)
    return (output_embeds, vq_loss)


# speedup vs baseline: 1.2256x; 1.2256x over previous
import functools

import jax
import jax.numpy as jnp
from jax.experimental import pallas as pl
from jax.experimental.pallas import tpu as pltpu

_K = 32
_T = 128
_C = 512


def _fused_kernel(xe_ref, xl_ref, wenc_ref, benc_ref, wdec_ref, bdec_ref,
                  y_ref, psum_ref, z_ref, bits_ref, *, n_chunks, n_blocks, k):
    i = pl.program_id(0)
    j = pl.program_id(1)
    par = jax.lax.rem(i, 2)
    prv = jax.lax.rem(i + 1, 2)
    kf = jnp.float32(k)

    @pl.when(i < n_blocks)
    def _encode():
        zc = jnp.dot(xe_ref[...], wenc_ref[...],
                     preferred_element_type=jnp.float32)
        zc = jnp.maximum(zc + benc_ref[...], 0.0)
        z_ref[par, :, pl.ds(j * _C, _C)] = zc

    @pl.when(i >= 1)
    def _search():
        z = z_ref[prv]
        t_rows = z.shape[0]

        def count_ge(thr):
            return jnp.sum((z >= thr).astype(jnp.float32), axis=1,
                           keepdims=True)

        bits = jnp.where(j == 0, jnp.int32(0), bits_ref[...])
        cand = bits | (jnp.int32(1) << (30 - j))
        thr_c = jax.lax.bitcast_convert_type(cand, jnp.float32)
        bits = jnp.where(count_ge(thr_c) >= kf, cand, bits)
        bits_ref[...] = bits

        @pl.when(j == n_chunks - 1)
        def _finish():
            t0 = jax.lax.bitcast_convert_type(bits, jnp.float32)
            c0 = count_ge(t0)

            def p2_cond(state):
                _, c, done = state
                live = jnp.where((done < 0.5) & (c > kf), 1.0, 0.0)
                return jnp.max(live) > 0.5

            def p2_body(state):
                t, c, done = state
                mn = jnp.min(jnp.where(z >= t, z, jnp.float32(jnp.inf)),
                             axis=1, keepdims=True)
                nxt = jax.lax.bitcast_convert_type(
                    jax.lax.bitcast_convert_type(mn, jnp.int32) + 1,
                    jnp.float32)
                c2 = count_ge(nxt)
                grp_done = c2 < kf
                active = (done < 0.5) & (c > kf)
                t_new = jnp.where(active, jnp.where(grp_done, mn, nxt), t)
                c_new = jnp.where(active & jnp.logical_not(grp_done), c2, c)
                done_new = jnp.where(active & grp_done, 1.0, done)
                return (t_new, c_new, done_new)

            thr, _, _ = jax.lax.while_loop(
                p2_cond, p2_body,
                (t0, c0, jnp.zeros((t_rows, 1), jnp.float32)))

            z_sparse = jnp.where(z >= thr, z, 0.0)
            y = jnp.dot(z_sparse, wdec_ref[...],
                        preferred_element_type=jnp.float32) + bdec_ref[...]
            y_ref[...] = y
            d = y - xl_ref[...]
            psum_ref[...] = jnp.broadcast_to(
                jnp.sum(d * d, keepdims=True).reshape(1, 1, 1), (1, 1, 128))


def kernel(inputs_embeds, W_enc, b_enc, W_dec, b_dec):
    bsz, seq, dim = inputs_embeds.shape
    code = W_enc.shape[1]
    tokens = bsz * seq
    n_blocks = tokens // _T
    n_chunks = code // _C

    x2 = inputs_embeds.reshape(tokens, dim)
    benc2 = b_enc.reshape(1, code)
    bdec2 = b_dec.reshape(1, dim)

    nb1 = n_blocks - 1
    grid = (n_blocks + 1, n_chunks)
    y, psums = pl.pallas_call(
        functools.partial(_fused_kernel, n_chunks=n_chunks,
                          n_blocks=n_blocks, k=_K),
        grid=grid,
        in_specs=[
            pl.BlockSpec((_T, dim), lambda i, j: (jax.lax.min(i, nb1), 0)),
            pl.BlockSpec((_T, dim), lambda i, j: (jax.lax.max(i - 1, 0), 0)),
            pl.BlockSpec((dim, _C), lambda i, j: (0, j)),
            pl.BlockSpec((1, _C), lambda i, j: (0, j)),
            pl.BlockSpec((code, dim), lambda i, j: (0, 0)),
            pl.BlockSpec((1, dim), lambda i, j: (0, 0)),
        ],
        out_specs=[
            pl.BlockSpec((_T, dim), lambda i, j: (jax.lax.max(i - 1, 0), 0)),
            pl.BlockSpec((1, 1, 128),
                         lambda i, j: (jax.lax.max(i - 1, 0), 0, 0)),
        ],
        out_shape=[
            jax.ShapeDtypeStruct((tokens, dim), jnp.float32),
            jax.ShapeDtypeStruct((n_blocks, 1, 128), jnp.float32),
        ],
        scratch_shapes=[
            pltpu.VMEM((2, _T, code), jnp.float32),
            pltpu.VMEM((_T, 1), jnp.int32),
        ],
        compiler_params=pltpu.CompilerParams(
            dimension_semantics=("arbitrary", "arbitrary"),
        ),
    )(x2, x2, W_enc, benc2, W_dec, bdec2)

    output_embeds = y.reshape(bsz, seq, dim)
    vq_loss = 0.25 * jnp.sum(psums[:, 0, 0]) / (tokens * dim)
    return (output_embeds, vq_loss)


# speedup vs baseline: 1.6728x; 1.3649x over previous
import functools

import jax
import jax.numpy as jnp
import numpy as np
from jax.experimental import pallas as pl
from jax.experimental.pallas import tpu as pltpu
from jax.sharding import Mesh, PartitionSpec as P

try:
    from jax.experimental.shard_map import shard_map as _shard_map
except ImportError:
    from jax import shard_map as _shard_map

_K = 32
_T = 128
_C = 512


def _fused_kernel(xe_ref, xl_ref, wenc_ref, benc_ref, wdec_ref, bdec_ref,
                  y_ref, psum_ref, z_ref, bits_ref, *, n_chunks, n_blocks, k):
    i = pl.program_id(0)
    j = pl.program_id(1)
    par = jax.lax.rem(i, 2)
    prv = jax.lax.rem(i + 1, 2)
    kf = jnp.float32(k)

    @pl.when(i < n_blocks)
    def _encode():
        zc = jnp.dot(xe_ref[...], wenc_ref[...],
                     preferred_element_type=jnp.float32)
        zc = jnp.maximum(zc + benc_ref[...], 0.0)
        z_ref[par, :, pl.ds(j * _C, _C)] = zc

    @pl.when(i >= 1)
    def _search():
        z = z_ref[prv]
        t_rows = z.shape[0]

        def count_ge(thr):
            return jnp.sum((z >= thr).astype(jnp.float32), axis=1,
                           keepdims=True)

        bits = jnp.where(j == 0, jnp.int32(0), bits_ref[...])
        cand = bits | (jnp.int32(1) << (30 - j))
        thr_c = jax.lax.bitcast_convert_type(cand, jnp.float32)
        bits = jnp.where(count_ge(thr_c) >= kf, cand, bits)
        bits_ref[...] = bits

        @pl.when(j == n_chunks - 1)
        def _finish():
            t0 = jax.lax.bitcast_convert_type(bits, jnp.float32)
            c0 = count_ge(t0)

            def p2_cond(state):
                _, c, done = state
                live = jnp.where((done < 0.5) & (c > kf), 1.0, 0.0)
                return jnp.max(live) > 0.5

            def p2_body(state):
                t, c, done = state
                mn = jnp.min(jnp.where(z >= t, z, jnp.float32(jnp.inf)),
                             axis=1, keepdims=True)
                nxt = jax.lax.bitcast_convert_type(
                    jax.lax.bitcast_convert_type(mn, jnp.int32) + 1,
                    jnp.float32)
                c2 = count_ge(nxt)
                grp_done = c2 < kf
                active = (done < 0.5) & (c > kf)
                t_new = jnp.where(active, jnp.where(grp_done, mn, nxt), t)
                c_new = jnp.where(active & jnp.logical_not(grp_done), c2, c)
                done_new = jnp.where(active & grp_done, 1.0, done)
                return (t_new, c_new, done_new)

            thr, _, _ = jax.lax.while_loop(
                p2_cond, p2_body,
                (t0, c0, jnp.zeros((t_rows, 1), jnp.float32)))

            z_sparse = jnp.where(z >= thr, z, 0.0)
            y = jnp.dot(z_sparse, wdec_ref[...],
                        preferred_element_type=jnp.float32) + bdec_ref[...]
            y_ref[...] = y
            d = y - xl_ref[...]
            psum_ref[...] = jnp.broadcast_to(
                jnp.sum(d * d, keepdims=True).reshape(1, 1, 1), (1, 1, 128))


def _forward(x2, W_enc, benc2, W_dec, bdec2):
    tokens, dim = x2.shape
    code = W_enc.shape[1]
    n_blocks = tokens // _T
    n_chunks = code // _C

    nb1 = n_blocks - 1
    grid = (n_blocks + 1, n_chunks)
    y, psums = pl.pallas_call(
        functools.partial(_fused_kernel, n_chunks=n_chunks,
                          n_blocks=n_blocks, k=_K),
        grid=grid,
        in_specs=[
            pl.BlockSpec((_T, dim), lambda i, j: (jax.lax.min(i, nb1), 0)),
            pl.BlockSpec((_T, dim), lambda i, j: (jax.lax.max(i - 1, 0), 0)),
            pl.BlockSpec((dim, _C), lambda i, j: (0, j)),
            pl.BlockSpec((1, _C), lambda i, j: (0, j)),
            pl.BlockSpec((code, dim), lambda i, j: (0, 0)),
            pl.BlockSpec((1, dim), lambda i, j: (0, 0)),
        ],
        out_specs=[
            pl.BlockSpec((_T, dim), lambda i, j: (jax.lax.max(i - 1, 0), 0)),
            pl.BlockSpec((1, 1, 128),
                         lambda i, j: (jax.lax.max(i - 1, 0), 0, 0)),
        ],
        out_shape=[
            jax.ShapeDtypeStruct((tokens, dim), jnp.float32),
            jax.ShapeDtypeStruct((n_blocks, 1, 128), jnp.float32),
        ],
        scratch_shapes=[
            pltpu.VMEM((2, _T, code), jnp.float32),
            pltpu.VMEM((_T, 1), jnp.int32),
        ],
        compiler_params=pltpu.CompilerParams(
            dimension_semantics=("arbitrary", "arbitrary"),
        ),
    )(x2, x2, W_enc, benc2, W_dec, bdec2)
    return y, psums


def kernel(inputs_embeds, W_enc, b_enc, W_dec, b_dec):
    bsz, seq, dim = inputs_embeds.shape
    code = W_enc.shape[1]
    tokens = bsz * seq

    x2 = inputs_embeds.reshape(tokens, dim)
    benc2 = b_enc.reshape(1, code)
    bdec2 = b_dec.reshape(1, dim)

    devs = jax.devices()
    n_shards = 2 if len(devs) >= 2 and tokens % (2 * _T) == 0 else 1
    if n_shards > 1:
        mesh = Mesh(np.array(devs[:n_shards]), ("d",))
        y, psums = _shard_map(
            _forward, mesh=mesh,
            in_specs=(P("d", None), P(None, None), P(None, None),
                      P(None, None), P(None, None)),
            out_specs=(P("d", None), P("d", None, None)),
            check_rep=False,
        )(x2, W_enc, benc2, W_dec, bdec2)
    else:
        y, psums = _forward(x2, W_enc, benc2, W_dec, bdec2)

    output_embeds = y.reshape(bsz, seq, dim)
    vq_loss = 0.25 * jnp.sum(psums[:, 0, 0]) / (tokens * dim)
    return (output_embeds, vq_loss)
